# baseline (speedup 1.0000x reference)
import jax
import jax.numpy as jnp
import numpy as np
from jax import lax
from jax.experimental import pallas as pl
from jax.experimental.pallas import tpu as pltpu
from jax.experimental.pallas import tpu_sc as plsc

D = 64
_TRIL = D * (D + 1) // 2
_XW = D + _TRIL
_NC, _NS = 2, 16
_NW = _NC * _NS
_R = 8
_CHUNKS = _TRIL // 16


def _sc_body(x_hbm, idx_hbm, loc_hbm, out_hbm, x_v, idx_v, loc_v, out_v):
    c = lax.axis_index("c")
    s = lax.axis_index("s")
    wid = s * _NC + c
    rows_per_w = x_hbm.shape[0] // (_XW * _NW)
    groups = rows_per_w // _R
    base0 = wid * rows_per_w

    pltpu.sync_copy(idx_hbm, idx_v)

    zeros16 = jnp.zeros((16,), jnp.float32)

    def zero_body(j, carry):
        out_v[pl.ds(j * 16, 16)] = zeros16
        return carry

    lax.fori_loop(0, _R * D * D // 16, zero_body, 0)

    def group_body(g, carry):
        base = base0 + g * _R
        pltpu.sync_copy(x_hbm.at[pl.ds(base * _XW, _R * _XW)], x_v)

        def chunk_body(k, carry2):
            dst = idx_v[pl.ds(k * 16, 16)]
            for rr in range(_R):
                v = x_v[pl.ds(rr * _XW + D + k * 16, 16)]
                plsc.store_scatter(out_v, [dst + rr * (D * D)], v)
            return carry2

        lax.fori_loop(0, _CHUNKS, chunk_body, 0)

        for cc in range(4):
            i = lax.iota(jnp.int32, 16) + cc * 16
            dsrc = ((i * (i + 3)) >> 1) + D
            ddst = i * (D + 1)
            for rr in range(_R):
                gv = plsc.load_gather(x_v, [dsrc + rr * _XW])
                plsc.store_scatter(out_v, [ddst + rr * (D * D)], jnp.exp(gv))

        for rr in range(_R):
            for cc in range(4):
                loc_v[pl.ds(rr * D + cc * 16, 16)] = (
                    x_v[pl.ds(rr * _XW + cc * 16, 16)])

        pltpu.sync_copy(out_v, out_hbm.at[pl.ds(base * D * D, _R * D * D)])
        pltpu.sync_copy(loc_v, loc_hbm.at[pl.ds(base * D, _R * D)])
        return carry

    lax.fori_loop(0, groups, group_body, 0)


def kernel(x):
    B = x.shape[0]
    ti, tj = np.tril_indices(D)
    idx = jnp.asarray((ti * D + tj).astype(np.int32))

    mesh = plsc.VectorSubcoreMesh(
        core_axis_name="c", subcore_axis_name="s",
        num_cores=_NC, num_subcores=_NS)
    run = pl.kernel(
        _sc_body,
        out_type=[
            jax.ShapeDtypeStruct((B * D,), jnp.float32),
            jax.ShapeDtypeStruct((B * D * D,), jnp.float32),
        ],
        mesh=mesh,
        scratch_types=[
            pltpu.VMEM((_R * _XW,), jnp.float32),
            pltpu.VMEM((_TRIL,), jnp.int32),
            pltpu.VMEM((_R * D,), jnp.float32),
            pltpu.VMEM((_R * D * D,), jnp.float32),
        ],
        compiler_params=pltpu.CompilerParams(needs_layout_passes=False),
    )
    loc, flat = run(x.reshape(B * _XW), idx)
    return loc.reshape(B, D), flat.reshape(B, D, D)


# speedup vs baseline: 1.1472x; 1.1472x over previous
import jax
import jax.numpy as jnp
import numpy as np
from jax import lax
from jax.experimental import pallas as pl
from jax.experimental.pallas import tpu as pltpu
from jax.experimental.pallas import tpu_sc as plsc

D = 64
_TRIL = D * (D + 1) // 2
_XW = D + _TRIL
_NC, _NS = 2, 16
_NW = _NC * _NS
_R = 8
_CHUNKS = _TRIL // 16


def _sc_body(x_hbm, idx_hbm, loc_hbm, out_hbm,
             x_v0, x_v1, idx_v, loc_v0, loc_v1, out_v0, out_v1,
             in_s0, in_s1, out_s0, out_s1, loc_s0, loc_s1):
    c = lax.axis_index("c")
    s = lax.axis_index("s")
    wid = s * _NC + c
    rows_per_w = x_hbm.shape[0] // (_XW * _NW)
    groups = rows_per_w // _R
    base0 = wid * rows_per_w

    x_v = [x_v0, x_v1]
    loc_v = [loc_v0, loc_v1]
    out_v = [out_v0, out_v1]
    in_s = [in_s0, in_s1]
    out_s = [out_s0, out_s1]
    loc_s = [loc_s0, loc_s1]

    pltpu.sync_copy(idx_hbm, idx_v)

    zeros16 = jnp.zeros((16,), jnp.float32)

    def zero_body(j, carry):
        out_v0[pl.ds(j * 16, 16)] = zeros16
        out_v1[pl.ds(j * 16, 16)] = zeros16
        return carry

    lax.fori_loop(0, _R * D * D // 16, zero_body, 0)

    def in_copy(g, b):
        base = base0 + g * _R
        return pltpu.make_async_copy(
            x_hbm.at[pl.ds(base * _XW, _R * _XW)], x_v[b], in_s[b])

    def out_copy(g, b):
        base = base0 + g * _R
        return pltpu.make_async_copy(
            out_v[b], out_hbm.at[pl.ds(base * D * D, _R * D * D)], out_s[b])

    def loc_copy(g, b):
        base = base0 + g * _R
        return pltpu.make_async_copy(
            loc_v[b], loc_hbm.at[pl.ds(base * D, _R * D)], loc_s[b])

    in_copy(0, 0).start()
    in_copy(1, 1).start()

    def compute(g, b):
        xb, ob, lb = x_v[b], out_v[b], loc_v[b]

        def chunk_body(k, carry2):
            dst = idx_v[pl.ds(k * 16, 16)]
            for rr in range(_R):
                v = xb[pl.ds(rr * _XW + D + k * 16, 16)]
                plsc.store_scatter(ob, [dst + rr * (D * D)], v)
            return carry2

        lax.fori_loop(0, _CHUNKS, chunk_body, 0)

        for cc in range(4):
            i = lax.iota(jnp.int32, 16) + cc * 16
            dsrc = ((i * (i + 3)) >> 1) + D
            ddst = i * (D + 1)
            for rr in range(_R):
                gv = plsc.load_gather(xb, [dsrc + rr * _XW])
                plsc.store_scatter(ob, [ddst + rr * (D * D)], jnp.exp(gv))

        for rr in range(_R):
            for cc in range(4):
                lb[pl.ds(rr * D + cc * 16, 16)] = (
                    xb[pl.ds(rr * _XW + cc * 16, 16)])

    def group_body(gg, carry):
        for b in range(2):
            g = gg * 2 + b
            in_copy(g, b).wait()

            @pl.when(g >= 2)
            def _():
                out_copy(g - 2, b).wait()
                loc_copy(g - 2, b).wait()

            compute(g, b)
            out_copy(g, b).start()
            loc_copy(g, b).start()

            @pl.when(g + 2 < groups)
            def _():
                in_copy(g + 2, b).start()
        return carry

    lax.fori_loop(0, groups // 2, group_body, 0)

    for b in range(2):
        g = groups - 2 + b
        out_copy(g, b).wait()
        loc_copy(g, b).wait()


def kernel(x):
    B = x.shape[0]
    ti, tj = np.tril_indices(D)
    idx = jnp.asarray((ti * D + tj).astype(np.int32))

    mesh = plsc.VectorSubcoreMesh(
        core_axis_name="c", subcore_axis_name="s",
        num_cores=_NC, num_subcores=_NS)
    run = pl.kernel(
        _sc_body,
        out_type=[
            jax.ShapeDtypeStruct((B * D,), jnp.float32),
            jax.ShapeDtypeStruct((B * D * D,), jnp.float32),
        ],
        mesh=mesh,
        scratch_types=[
            pltpu.VMEM((_R * _XW,), jnp.float32),
            pltpu.VMEM((_R * _XW,), jnp.float32),
            pltpu.VMEM((_TRIL,), jnp.int32),
            pltpu.VMEM((_R * D,), jnp.float32),
            pltpu.VMEM((_R * D,), jnp.float32),
            pltpu.VMEM((_R * D * D,), jnp.float32),
            pltpu.VMEM((_R * D * D,), jnp.float32),
            pltpu.SemaphoreType.DMA,
            pltpu.SemaphoreType.DMA,
            pltpu.SemaphoreType.DMA,
            pltpu.SemaphoreType.DMA,
            pltpu.SemaphoreType.DMA,
            pltpu.SemaphoreType.DMA,
        ],
        compiler_params=pltpu.CompilerParams(needs_layout_passes=False),
    )
    loc, flat = run(x.reshape(B * _XW), idx)
    return loc.reshape(B, D), flat.reshape(B, D, D)
